# baseline (speedup 1.0000x reference)
import jax
import jax.numpy as jnp
from jax.experimental import pallas as pl
from jax.experimental.pallas import tpu as pltpu

_N = 10000
_A = 128
_B = 128
_BM = 200


def _hw_kernel(h_ref, w_ref, out_ref):
    out_ref[...] = jnp.dot(h_ref[...], w_ref[...],
                           preferred_element_type=jnp.float32)


def _ahw_kernel(a_ref, hw_ref, bias_ref, out_ref):
    acc = jnp.dot(a_ref[...], hw_ref[...], preferred_element_type=jnp.float32)
    out_ref[...] = acc + bias_ref[...]


def kernel(structure, H, W, bias):
    hw = pl.pallas_call(
        _hw_kernel,
        out_shape=jax.ShapeDtypeStruct((_N, _B), jnp.float32),
        grid=(10,),
        in_specs=[
            pl.BlockSpec((_N // 10, _A), lambda i: (i, 0)),
            pl.BlockSpec((_A, _B), lambda i: (0, 0)),
        ],
        out_specs=pl.BlockSpec((_N // 10, _B), lambda i: (i, 0)),
    )(H, W)

    out = pl.pallas_call(
        _ahw_kernel,
        out_shape=jax.ShapeDtypeStruct((_N, _B), jnp.float32),
        grid=(_N // _BM,),
        in_specs=[
            pl.BlockSpec((_BM, _N), lambda i: (i, 0)),
            pl.BlockSpec((_N, _B), lambda i: (0, 0)),
            pl.BlockSpec((1, _B), lambda i: (0, 0)),
        ],
        out_specs=pl.BlockSpec((_BM, _B), lambda i: (i, 0)),
        compiler_params=pltpu.CompilerParams(
            dimension_semantics=("arbitrary",),
        ),
    )(structure, hw, bias.reshape(1, _B))
    return out


# speedup vs baseline: 1.0374x; 1.0374x over previous
import jax
import jax.numpy as jnp
from jax.experimental import pallas as pl
from jax.experimental.pallas import tpu as pltpu

_N = 10000
_A = 128
_B = 128
_BM = 200


def _fused_kernel(h_ref, w_ref, a_ref, bias_ref, out_ref, hw_ref):
    @pl.when(pl.program_id(0) == 0)
    def _():
        hw_ref[...] = jnp.dot(h_ref[...], w_ref[...],
                              preferred_element_type=jnp.float32,
                              precision=jax.lax.Precision.HIGHEST)

    acc = jnp.dot(a_ref[...], hw_ref[...],
                  preferred_element_type=jnp.float32,
                  precision=jax.lax.Precision.DEFAULT)
    out_ref[...] = acc + bias_ref[...]


def kernel(structure, H, W, bias):
    return pl.pallas_call(
        _fused_kernel,
        out_shape=jax.ShapeDtypeStruct((_N, _B), jnp.float32),
        grid=(_N // _BM,),
        in_specs=[
            pl.BlockSpec((_N, _A), lambda i: (0, 0)),
            pl.BlockSpec((_A, _B), lambda i: (0, 0)),
            pl.BlockSpec((_BM, _N), lambda i: (i, 0)),
            pl.BlockSpec((1, _B), lambda i: (0, 0)),
        ],
        out_specs=pl.BlockSpec((_BM, _B), lambda i: (i, 0)),
        scratch_shapes=[pltpu.VMEM((_N, _B), jnp.float32)],
        compiler_params=pltpu.CompilerParams(
            dimension_semantics=("arbitrary",),
        ),
    )(H, W, structure, bias.reshape(1, _B))
